# baseline (speedup 1.0000x reference)
import functools

import jax
import jax.numpy as jnp
from jax import lax
from jax.experimental import pallas as pl
from jax.experimental.pallas import tpu as pltpu
from jax.experimental.pallas import tpu_sc as plsc

N_NODES = 10000
D_IN = 128
D_OUT = 128
N_EDGES = 320000
BN_EPS = 1e-5

NC = 2
NS = 16
U = 64
EPT = N_EDGES // (NC * NS)
UPT = EPT // U
ETAIL = EPT - UPT * U
URPAD = 160
SR = 32
NST = URPAD // SR
RING = 4
ZROWS = 25
ROWS_PT = N_NODES // NS
NBCH = N_NODES // U
BTAIL = N_NODES - NBCH * U


def _sc_agg_gather(f, tgt, src, bat_main, bat_tail):
    mesh = plsc.VectorSubcoreMesh(core_axis_name="c", subcore_axis_name="s")

    @functools.partial(
        pl.kernel,
        out_type=[jax.ShapeDtypeStruct((N_NODES, D_IN), jnp.float32)] * 3,
        mesh=mesh,
        scratch_types=[
            pltpu.VMEM_SHARED((N_NODES, D_IN), jnp.float32),
            pltpu.VMEM((2, SR, U), jnp.int32),
            pltpu.VMEM((2, SR, U), jnp.int32),
            pltpu.VMEM((ETAIL,), jnp.int32),
            pltpu.VMEM((ETAIL,), jnp.int32),
            pltpu.VMEM((RING, U, D_IN), jnp.float32),
            pltpu.VMEM((2, U), jnp.int32),
            pltpu.VMEM((BTAIL,), jnp.int32),
            pltpu.SemaphoreType.DMA,
        ],
    )
    def sc_kernel(f_h, tgt_h, src_h, batm_h, batt_h,
                  hf_h, haa_h, hab_h,
                  acc, tgt_v, src_v, ttidx, stidx, rows, bidx, bidx_t, sem):
        c = lax.axis_index("c")
        s = lax.axis_index("s")
        w = c * NS + s

        zv = jnp.zeros((16,), jnp.float32)
        for r in range(ZROWS):
            for k in range(D_IN // 16):
                rows[0, r, pl.ds(k * 16, 16)] = zv
        for k in range(ROWS_PT // ZROWS):
            pltpu.sync_copy(rows.at[0, pl.ds(0, ZROWS)],
                            acc.at[pl.ds(s * ROWS_PT + k * ZROWS, ZROWS)])
        plsc.subcore_barrier()

        def _issue(x):
            pltpu.async_copy(
                f_h.at[tgt_v.at[lax.rem(lax.div(x, SR), 2), lax.rem(x, SR)]],
                rows.at[lax.rem(x, RING)], sem)

        def _wait(x):
            pltpu.make_async_copy(
                f_h.at[tgt_v.at[lax.rem(lax.div(x, SR), 2), lax.rem(x, SR)]],
                rows.at[lax.rem(x, RING)], sem).wait()

        pltpu.sync_copy(tgt_h.at[c, s, pl.ds(0, SR)], tgt_v.at[0])
        pltpu.sync_copy(src_h.at[c, s, pl.ds(0, SR)], src_v.at[0])
        for x in range(RING - 1):
            _issue(x)

        def edge_body(u, carry):
            k = lax.div(u, SR)

            @pl.when((lax.rem(u, SR) == RING - 1) & (k + 1 < NST))
            def _():
                pltpu.sync_copy(
                    tgt_h.at[c, s, pl.ds((k + 1) * SR, SR)],
                    tgt_v.at[lax.rem(k + 1, 2)])
                pltpu.sync_copy(
                    src_h.at[c, s, pl.ds((k + 1) * SR, SR)],
                    src_v.at[lax.rem(k + 1, 2)])

            _wait(u)
            pltpu.sync_copy(rows.at[lax.rem(u, RING)],
                            acc.at[src_v.at[lax.rem(k, 2), lax.rem(u, SR)]],
                            add=True)

            @pl.when(u + RING - 1 < UPT)
            def _():
                _issue(u + RING - 1)

            return carry

        lax.fori_loop(0, UPT, edge_body, 0)

        pltpu.sync_copy(tgt_h.at[c, s, UPT, pl.ds(0, ETAIL)], ttidx)
        pltpu.sync_copy(src_h.at[c, s, UPT, pl.ds(0, ETAIL)], stidx)
        pltpu.async_copy(f_h.at[ttidx], rows.at[0, pl.ds(0, ETAIL)], sem).wait()
        pltpu.sync_copy(rows.at[0, pl.ds(0, ETAIL)], acc.at[stidx], add=True)
        plsc.subcore_barrier()

        nb_f = (NBCH - w + NC * NS - 1) // (NC * NS)

        @pl.when(nb_f > 0)
        def _():
            pltpu.sync_copy(batm_h.at[w], bidx.at[0])
            pltpu.async_copy(f_h.at[bidx.at[0]], rows.at[0], sem)

            def bf_body(i, carry):
                p = lax.rem(i, 2)
                j = w + NC * NS * i
                pltpu.make_async_copy(
                    f_h.at[bidx.at[p]], rows.at[p], sem).wait()

                @pl.when(i + 1 < nb_f)
                def _():
                    pltpu.sync_copy(batm_h.at[j + NC * NS], bidx.at[1 - p])
                    pltpu.async_copy(
                        f_h.at[bidx.at[1 - p]], rows.at[1 - p], sem)

                pltpu.sync_copy(rows.at[p], hf_h.at[pl.ds(j * U, U)])
                return carry

            lax.fori_loop(0, nb_f, bf_body, 0)

        @pl.when((c == 0) & (s == NS - 2))
        def _():
            pltpu.sync_copy(batt_h, bidx_t)
            pltpu.async_copy(
                f_h.at[bidx_t], rows.at[0, pl.ds(0, BTAIL)], sem).wait()
            pltpu.sync_copy(rows.at[0, pl.ds(0, BTAIL)],
                            hf_h.at[pl.ds(NBCH * U, BTAIL)])

        nb_a = (NBCH - s + NS - 1) // NS

        def ba_body(i, carry):
            p = lax.rem(i, 2)
            j = s + NS * i
            pltpu.make_async_copy(acc.at[bidx.at[p]], rows.at[p], sem).wait()

            @pl.when(i + 1 < nb_a)
            def _():
                pltpu.sync_copy(batm_h.at[j + NS], bidx.at[1 - p])
                pltpu.async_copy(acc.at[bidx.at[1 - p]], rows.at[1 - p], sem)

            @pl.when(c == 0)
            def _():
                pltpu.sync_copy(rows.at[p], haa_h.at[pl.ds(j * U, U)])

            @pl.when(c == 1)
            def _():
                pltpu.sync_copy(rows.at[p], hab_h.at[pl.ds(j * U, U)])

            return carry

        pltpu.sync_copy(batm_h.at[s], bidx.at[0])
        pltpu.async_copy(acc.at[bidx.at[0]], rows.at[0], sem)
        lax.fori_loop(0, nb_a, ba_body, 0)

        @pl.when(s == NS - 1)
        def _():
            pltpu.sync_copy(batt_h, bidx_t)
            pltpu.async_copy(
                acc.at[bidx_t], rows.at[0, pl.ds(0, BTAIL)], sem).wait()

            @pl.when(c == 0)
            def _():
                pltpu.sync_copy(rows.at[0, pl.ds(0, BTAIL)],
                                haa_h.at[pl.ds(NBCH * U, BTAIL)])

            @pl.when(c == 1)
            def _():
                pltpu.sync_copy(rows.at[0, pl.ds(0, BTAIL)],
                                hab_h.at[pl.ds(NBCH * U, BTAIL)])

    return sc_kernel(f, tgt, src, bat_main, bat_tail)


_RB = 1000


def _tc_dense_body(hf, haa, hab, w, b, g, bt, o):
    dn = (((1,), (1,)), ((), ()))
    ha = haa[...] + hab[...]
    x = lax.dot_general(hf[...], w[...][:, :D_IN], dn,
                        preferred_element_type=jnp.float32)
    x += lax.dot_general(ha, w[...][:, D_IN:], dn,
                         preferred_element_type=jnp.float32)
    z = jnp.maximum(x + b[...], 0.0)
    scale = g[...] * lax.rsqrt(jnp.float32(1.0 + BN_EPS))
    z = z * scale + bt[...]
    nrm = jnp.sqrt(jnp.sum(z * z, axis=1, keepdims=True))
    o[...] = z / (nrm + 1e-6)


def _tc_dense(hf, haa, hab, w, b, g, bt):
    grid = (N_NODES // _RB,)
    row_spec = pl.BlockSpec((_RB, D_IN), lambda i: (i, 0))
    vec_spec = pl.BlockSpec((1, D_OUT), lambda i: (0, 0))
    return pl.pallas_call(
        _tc_dense_body,
        grid=grid,
        in_specs=[row_spec, row_spec, row_spec,
                  pl.BlockSpec((D_OUT, 2 * D_IN), lambda i: (0, 0)),
                  vec_spec, vec_spec, vec_spec],
        out_specs=pl.BlockSpec((_RB, D_OUT), lambda i: (i, 0)),
        out_shape=jax.ShapeDtypeStruct((N_NODES, D_OUT), jnp.float32),
    )(hf, haa, hab, w, b, g, bt)


def kernel(features, batch, edge_index, W, b, gamma, beta):
    f32 = jnp.float32
    i32 = jnp.int32
    padcols = URPAD * U - EPT
    per_tile_src = edge_index[0].astype(i32).reshape(NC * NS, EPT)
    per_tile_tgt = edge_index[1].astype(i32).reshape(NC * NS, EPT)
    src_r = jnp.pad(per_tile_src, ((0, 0), (0, padcols))).reshape(
        NC, NS, URPAD, U)
    tgt_r = jnp.pad(per_tile_tgt, ((0, 0), (0, padcols))).reshape(
        NC, NS, URPAD, U)
    bat = batch.astype(i32)
    bat_main = bat[:NBCH * U].reshape(NBCH, U)
    bat_tail = bat[NBCH * U:]
    hf, haa, hab = _sc_agg_gather(features, tgt_r, src_r, bat_main, bat_tail)
    return _tc_dense(hf, haa, hab,
                     W.astype(f32), b.reshape(1, D_OUT).astype(f32),
                     gamma.reshape(1, D_OUT).astype(f32),
                     beta.reshape(1, D_OUT).astype(f32))


# speedup vs baseline: 1.0382x; 1.0382x over previous
import functools

import jax
import jax.numpy as jnp
from jax import lax
from jax.experimental import pallas as pl
from jax.experimental.pallas import tpu as pltpu
from jax.experimental.pallas import tpu_sc as plsc

N_NODES = 10000
D_IN = 128
D_OUT = 128
N_EDGES = 320000
BN_EPS = 1e-5

NC = 2
NS = 16
U = 64
EPT = N_EDGES // (NC * NS)
UPT = EPT // U
ETAIL = EPT - UPT * U
URPAD = 160
SR = 32
NST = URPAD // SR
RING = 4
ROWS_PT = N_NODES // NS
ZFULL = ROWS_PT // U
ZREM = ROWS_PT - ZFULL * U
NBCH = N_NODES // U
BTAIL = N_NODES - NBCH * U


def _sc_agg_gather(f, tgt, src, bat_main, bat_tail):
    mesh = plsc.VectorSubcoreMesh(core_axis_name="c", subcore_axis_name="s")

    @functools.partial(
        pl.kernel,
        out_type=[jax.ShapeDtypeStruct((N_NODES, D_IN), jnp.float32)] * 3,
        mesh=mesh,
        scratch_types=[
            pltpu.VMEM_SHARED((N_NODES, D_IN), jnp.float32),
            pltpu.VMEM((2, SR, U), jnp.int32),
            pltpu.VMEM((2, SR, U), jnp.int32),
            pltpu.VMEM((ETAIL,), jnp.int32),
            pltpu.VMEM((ETAIL,), jnp.int32),
            pltpu.VMEM((RING, U, D_IN), jnp.float32),
            pltpu.VMEM((RING, U), jnp.int32),
            pltpu.VMEM((BTAIL,), jnp.int32),
            pltpu.SemaphoreType.DMA,
            pltpu.SemaphoreType.DMA,
        ],
    )
    def sc_kernel(f_h, tgt_h, src_h, batm_h, batt_h,
                  hf_h, haa_h, hab_h,
                  acc, tgt_v, src_v, ttidx, stidx, rows, bidx, bidx_t,
                  sem, semw):
        c = lax.axis_index("c")
        s = lax.axis_index("s")
        w = c * NS + s
        zbase = s * ROWS_PT

        zv = jnp.zeros((16,), jnp.float32)
        for q in range(RING):
            for r in range(U):
                for k in range(D_IN // 16):
                    rows[q, r, pl.ds(k * 16, 16)] = zv
        for z in range(ZFULL):
            pltpu.async_copy(rows.at[z % RING],
                             acc.at[pl.ds(zbase + z * U, U)], semw)
        pltpu.async_copy(rows.at[ZFULL % RING, pl.ds(0, ZREM)],
                         acc.at[pl.ds(zbase + ZFULL * U, ZREM)], semw)

        pltpu.sync_copy(tgt_h.at[c, s, pl.ds(0, SR)], tgt_v.at[0])
        pltpu.sync_copy(src_h.at[c, s, pl.ds(0, SR)], src_v.at[0])
        for z in range(ZFULL):
            pltpu.make_async_copy(rows.at[z % RING],
                                  acc.at[pl.ds(zbase + z * U, U)],
                                  semw).wait()
        pltpu.make_async_copy(rows.at[ZFULL % RING, pl.ds(0, ZREM)],
                              acc.at[pl.ds(zbase + ZFULL * U, ZREM)],
                              semw).wait()

        def _issue(x):
            pltpu.async_copy(
                f_h.at[tgt_v.at[lax.rem(lax.div(x, SR), 2), lax.rem(x, SR)]],
                rows.at[lax.rem(x, RING)], sem)

        def _wait(x):
            pltpu.make_async_copy(
                f_h.at[tgt_v.at[lax.rem(lax.div(x, SR), 2), lax.rem(x, SR)]],
                rows.at[lax.rem(x, RING)], sem).wait()

        for x in range(RING - 1):
            _issue(x)
        plsc.subcore_barrier()

        def edge_body(u, carry):
            k = lax.div(u, SR)

            @pl.when((lax.rem(u, SR) == RING - 1) & (k + 1 < NST))
            def _():
                pltpu.sync_copy(
                    tgt_h.at[c, s, pl.ds((k + 1) * SR, SR)],
                    tgt_v.at[lax.rem(k + 1, 2)])
                pltpu.sync_copy(
                    src_h.at[c, s, pl.ds((k + 1) * SR, SR)],
                    src_v.at[lax.rem(k + 1, 2)])

            _wait(u)
            pltpu.sync_copy(rows.at[lax.rem(u, RING)],
                            acc.at[src_v.at[lax.rem(k, 2), lax.rem(u, SR)]],
                            add=True)

            @pl.when(u + RING - 1 < UPT)
            def _():
                _issue(u + RING - 1)

            return carry

        lax.fori_loop(0, UPT, edge_body, 0)

        pltpu.sync_copy(tgt_h.at[c, s, UPT, pl.ds(0, ETAIL)], ttidx)
        pltpu.sync_copy(src_h.at[c, s, UPT, pl.ds(0, ETAIL)], stidx)
        pltpu.async_copy(f_h.at[ttidx], rows.at[0, pl.ds(0, ETAIL)], sem).wait()
        pltpu.sync_copy(rows.at[0, pl.ds(0, ETAIL)], acc.at[stidx], add=True)
        plsc.subcore_barrier()

        def _batch_ring(n, stride, first, src_ref, dst_ref):
            def _stage(i):
                pltpu.sync_copy(batm_h.at[first + stride * i],
                                bidx.at[lax.rem(i, RING)])

            def _g(i):
                pltpu.async_copy(src_ref.at[bidx.at[lax.rem(i, RING)]],
                                 rows.at[lax.rem(i, RING)], sem)

            def _gwait(i):
                pltpu.make_async_copy(src_ref.at[bidx.at[lax.rem(i, RING)]],
                                      rows.at[lax.rem(i, RING)], sem).wait()

            def _wr(i):
                pltpu.async_copy(
                    rows.at[lax.rem(i, RING)],
                    dst_ref.at[pl.ds((first + stride * i) * U, U)], semw)

            def _wrwait(i):
                pltpu.make_async_copy(
                    rows.at[lax.rem(i, RING)],
                    dst_ref.at[pl.ds((first + stride * i) * U, U)],
                    semw).wait()

            for i in range(RING - 1):
                _stage(i)
                _g(i)

            def body(i, carry):
                _gwait(i)
                _wr(i)

                @pl.when(i + RING - 1 < n)
                def _():
                    @pl.when(i >= 1)
                    def _():
                        _wrwait(i - 1)

                    _stage(i + RING - 1)
                    _g(i + RING - 1)

                return carry

            lax.fori_loop(0, n, body, 0)

            def drain(i, carry):
                _wrwait(i)
                return carry

            lax.fori_loop(lax.max(n - RING, 0), n, drain, 0)

        nb_f = (NBCH - w + NC * NS - 1) // (NC * NS)
        _batch_ring(nb_f, NC * NS, w, f_h, hf_h)

        @pl.when((c == 0) & (s == NS - 2))
        def _():
            pltpu.sync_copy(batt_h, bidx_t)
            pltpu.async_copy(
                f_h.at[bidx_t], rows.at[0, pl.ds(0, BTAIL)], sem).wait()
            pltpu.sync_copy(rows.at[0, pl.ds(0, BTAIL)],
                            hf_h.at[pl.ds(NBCH * U, BTAIL)])

        nb_a = (NBCH - s + NS - 1) // NS

        @pl.when(c == 0)
        def _():
            _batch_ring(nb_a, NS, s, acc, haa_h)

        @pl.when(c == 1)
        def _():
            _batch_ring(nb_a, NS, s, acc, hab_h)

        @pl.when(s == NS - 1)
        def _():
            pltpu.sync_copy(batt_h, bidx_t)
            pltpu.async_copy(
                acc.at[bidx_t], rows.at[0, pl.ds(0, BTAIL)], sem).wait()

            @pl.when(c == 0)
            def _():
                pltpu.sync_copy(rows.at[0, pl.ds(0, BTAIL)],
                                haa_h.at[pl.ds(NBCH * U, BTAIL)])

            @pl.when(c == 1)
            def _():
                pltpu.sync_copy(rows.at[0, pl.ds(0, BTAIL)],
                                hab_h.at[pl.ds(NBCH * U, BTAIL)])

    return sc_kernel(f, tgt, src, bat_main, bat_tail)


_RB = 1000


def _tc_dense_body(hf, haa, hab, w, b, g, bt, o):
    dn = (((1,), (1,)), ((), ()))
    ha = haa[...] + hab[...]
    x = lax.dot_general(hf[...], w[...][:, :D_IN], dn,
                        preferred_element_type=jnp.float32)
    x += lax.dot_general(ha, w[...][:, D_IN:], dn,
                         preferred_element_type=jnp.float32)
    z = jnp.maximum(x + b[...], 0.0)
    scale = g[...] * lax.rsqrt(jnp.float32(1.0 + BN_EPS))
    z = z * scale + bt[...]
    nrm = jnp.sqrt(jnp.sum(z * z, axis=1, keepdims=True))
    o[...] = z / (nrm + 1e-6)


def _tc_dense(hf, haa, hab, w, b, g, bt):
    grid = (N_NODES // _RB,)
    row_spec = pl.BlockSpec((_RB, D_IN), lambda i: (i, 0))
    vec_spec = pl.BlockSpec((1, D_OUT), lambda i: (0, 0))
    return pl.pallas_call(
        _tc_dense_body,
        grid=grid,
        in_specs=[row_spec, row_spec, row_spec,
                  pl.BlockSpec((D_OUT, 2 * D_IN), lambda i: (0, 0)),
                  vec_spec, vec_spec, vec_spec],
        out_specs=pl.BlockSpec((_RB, D_OUT), lambda i: (i, 0)),
        out_shape=jax.ShapeDtypeStruct((N_NODES, D_OUT), jnp.float32),
    )(hf, haa, hab, w, b, g, bt)


def kernel(features, batch, edge_index, W, b, gamma, beta):
    f32 = jnp.float32
    i32 = jnp.int32
    padcols = URPAD * U - EPT
    per_tile_src = edge_index[0].astype(i32).reshape(NC * NS, EPT)
    per_tile_tgt = edge_index[1].astype(i32).reshape(NC * NS, EPT)
    src_r = jnp.pad(per_tile_src, ((0, 0), (0, padcols))).reshape(
        NC, NS, URPAD, U)
    tgt_r = jnp.pad(per_tile_tgt, ((0, 0), (0, padcols))).reshape(
        NC, NS, URPAD, U)
    bat = batch.astype(i32)
    bat_main = bat[:NBCH * U].reshape(NBCH, U)
    bat_tail = bat[NBCH * U:]
    hf, haa, hab = _sc_agg_gather(features, tgt_r, src_r, bat_main, bat_tail)
    return _tc_dense(hf, haa, hab,
                     W.astype(f32), b.reshape(1, D_OUT).astype(f32),
                     gamma.reshape(1, D_OUT).astype(f32),
                     beta.reshape(1, D_OUT).astype(f32))


# speedup vs baseline: 1.0387x; 1.0005x over previous
import functools

import jax
import jax.numpy as jnp
from jax import lax
from jax.experimental import pallas as pl
from jax.experimental.pallas import tpu as pltpu
from jax.experimental.pallas import tpu_sc as plsc

N_NODES = 10000
D_IN = 128
D_OUT = 128
N_EDGES = 320000
BN_EPS = 1e-5

NC = 2
NS = 16
U = 64
EPT = N_EDGES // (NC * NS)
UPT = EPT // U
ETAIL = EPT - UPT * U
URPAD = 160
SR = 32
NST = URPAD // SR
RING = 4
ROWS_PT = N_NODES // NS
ZFULL = ROWS_PT // U
ZREM = ROWS_PT - ZFULL * U
NBCH = N_NODES // U
BTAIL = N_NODES - NBCH * U


def _sc_agg_gather(f, tgt, src, bat_main, bat_tail):
    mesh = plsc.VectorSubcoreMesh(core_axis_name="c", subcore_axis_name="s")

    @functools.partial(
        pl.kernel,
        out_type=[jax.ShapeDtypeStruct((N_NODES, D_IN), jnp.float32)] * 3,
        mesh=mesh,
        scratch_types=[
            pltpu.VMEM_SHARED((N_NODES, D_IN), jnp.float32),
            pltpu.VMEM((2, SR, U), jnp.int32),
            pltpu.VMEM((2, SR, U), jnp.int32),
            pltpu.VMEM((ETAIL,), jnp.int32),
            pltpu.VMEM((ETAIL,), jnp.int32),
            pltpu.VMEM((RING, U, D_IN), jnp.float32),
            pltpu.VMEM((RING, U), jnp.int32),
            pltpu.VMEM((BTAIL,), jnp.int32),
            pltpu.SemaphoreType.DMA,
            pltpu.SemaphoreType.DMA,
        ],
    )
    def sc_kernel(f_h, tgt_h, src_h, batm_h, batt_h,
                  hf_h, haa_h, hab_h,
                  acc, tgt_v, src_v, ttidx, stidx, rows, bidx, bidx_t,
                  sem, semw):
        c = lax.axis_index("c")
        s = lax.axis_index("s")
        w = c * NS + s
        zbase = s * ROWS_PT

        zv = jnp.zeros((16,), jnp.float32)
        for q in range(RING):
            for r in range(U):
                for k in range(D_IN // 16):
                    rows[q, r, pl.ds(k * 16, 16)] = zv
        for z in range(ZFULL):
            pltpu.async_copy(rows.at[z % RING],
                             acc.at[pl.ds(zbase + z * U, U)], semw)
        pltpu.async_copy(rows.at[ZFULL % RING, pl.ds(0, ZREM)],
                         acc.at[pl.ds(zbase + ZFULL * U, ZREM)], semw)

        pltpu.sync_copy(tgt_h.at[c, s, pl.ds(0, SR)], tgt_v.at[0])
        pltpu.sync_copy(src_h.at[c, s, pl.ds(0, SR)], src_v.at[0])
        for z in range(ZFULL):
            pltpu.make_async_copy(rows.at[z % RING],
                                  acc.at[pl.ds(zbase + z * U, U)],
                                  semw).wait()
        pltpu.make_async_copy(rows.at[ZFULL % RING, pl.ds(0, ZREM)],
                              acc.at[pl.ds(zbase + ZFULL * U, ZREM)],
                              semw).wait()

        def _issue(x):
            pltpu.async_copy(
                f_h.at[tgt_v.at[lax.rem(lax.div(x, SR), 2), lax.rem(x, SR)]],
                rows.at[lax.rem(x, RING)], sem)

        def _wait(x):
            pltpu.make_async_copy(
                f_h.at[tgt_v.at[lax.rem(lax.div(x, SR), 2), lax.rem(x, SR)]],
                rows.at[lax.rem(x, RING)], sem).wait()

        for x in range(RING - 1):
            _issue(x)
        plsc.subcore_barrier()

        def edge_body(u, carry):
            k = lax.div(u, SR)

            @pl.when((lax.rem(u, SR) == RING - 1) & (k + 1 < NST))
            def _():
                pltpu.sync_copy(
                    tgt_h.at[c, s, pl.ds((k + 1) * SR, SR)],
                    tgt_v.at[lax.rem(k + 1, 2)])
                pltpu.sync_copy(
                    src_h.at[c, s, pl.ds((k + 1) * SR, SR)],
                    src_v.at[lax.rem(k + 1, 2)])

            _wait(u)
            pltpu.sync_copy(rows.at[lax.rem(u, RING)],
                            acc.at[src_v.at[lax.rem(k, 2), lax.rem(u, SR)]],
                            add=True)

            @pl.when(u + RING - 1 < UPT)
            def _():
                _issue(u + RING - 1)

            return carry

        lax.fori_loop(0, UPT, edge_body, 0)

        pltpu.sync_copy(tgt_h.at[c, s, UPT, pl.ds(0, ETAIL)], ttidx)
        pltpu.sync_copy(src_h.at[c, s, UPT, pl.ds(0, ETAIL)], stidx)
        pltpu.async_copy(f_h.at[ttidx], rows.at[0, pl.ds(0, ETAIL)], sem).wait()
        pltpu.sync_copy(rows.at[0, pl.ds(0, ETAIL)], acc.at[stidx], add=True)

        def _batch_ring(n, stride, first, src_ref, dst_ref):
            def _stage(i):
                pltpu.sync_copy(batm_h.at[first + stride * i],
                                bidx.at[lax.rem(i, RING)])

            def _g(i):
                pltpu.async_copy(src_ref.at[bidx.at[lax.rem(i, RING)]],
                                 rows.at[lax.rem(i, RING)], sem)

            def _gwait(i):
                pltpu.make_async_copy(src_ref.at[bidx.at[lax.rem(i, RING)]],
                                      rows.at[lax.rem(i, RING)], sem).wait()

            def _wr(i):
                pltpu.async_copy(
                    rows.at[lax.rem(i, RING)],
                    dst_ref.at[pl.ds((first + stride * i) * U, U)], semw)

            def _wrwait(i):
                pltpu.make_async_copy(
                    rows.at[lax.rem(i, RING)],
                    dst_ref.at[pl.ds((first + stride * i) * U, U)],
                    semw).wait()

            for i in range(RING - 1):
                _stage(i)
                _g(i)

            def body(i, carry):
                _gwait(i)
                _wr(i)

                @pl.when(i + RING - 1 < n)
                def _():
                    @pl.when(i >= 1)
                    def _():
                        _wrwait(i - 1)

                    _stage(i + RING - 1)
                    _g(i + RING - 1)

                return carry

            lax.fori_loop(0, n, body, 0)

            def drain(i, carry):
                _wrwait(i)
                return carry

            lax.fori_loop(lax.max(n - RING, 0), n, drain, 0)

        nb_f = (NBCH - w + NC * NS - 1) // (NC * NS)
        _batch_ring(nb_f, NC * NS, w, f_h, hf_h)

        @pl.when((c == 0) & (s == NS - 2))
        def _():
            pltpu.sync_copy(batt_h, bidx_t)
            pltpu.async_copy(
                f_h.at[bidx_t], rows.at[0, pl.ds(0, BTAIL)], sem).wait()
            pltpu.sync_copy(rows.at[0, pl.ds(0, BTAIL)],
                            hf_h.at[pl.ds(NBCH * U, BTAIL)])

        plsc.subcore_barrier()

        nb_a = (NBCH - s + NS - 1) // NS

        @pl.when(c == 0)
        def _():
            _batch_ring(nb_a, NS, s, acc, haa_h)

        @pl.when(c == 1)
        def _():
            _batch_ring(nb_a, NS, s, acc, hab_h)

        @pl.when(s == NS - 1)
        def _():
            pltpu.sync_copy(batt_h, bidx_t)
            pltpu.async_copy(
                acc.at[bidx_t], rows.at[0, pl.ds(0, BTAIL)], sem).wait()

            @pl.when(c == 0)
            def _():
                pltpu.sync_copy(rows.at[0, pl.ds(0, BTAIL)],
                                haa_h.at[pl.ds(NBCH * U, BTAIL)])

            @pl.when(c == 1)
            def _():
                pltpu.sync_copy(rows.at[0, pl.ds(0, BTAIL)],
                                hab_h.at[pl.ds(NBCH * U, BTAIL)])

    return sc_kernel(f, tgt, src, bat_main, bat_tail)


_RB = 1000


def _tc_dense_body(hf, haa, hab, w, b, g, bt, o):
    dn = (((1,), (1,)), ((), ()))
    ha = haa[...] + hab[...]
    x = lax.dot_general(hf[...], w[...][:, :D_IN], dn,
                        preferred_element_type=jnp.float32)
    x += lax.dot_general(ha, w[...][:, D_IN:], dn,
                         preferred_element_type=jnp.float32)
    z = jnp.maximum(x + b[...], 0.0)
    scale = g[...] * lax.rsqrt(jnp.float32(1.0 + BN_EPS))
    z = z * scale + bt[...]
    nrm = jnp.sqrt(jnp.sum(z * z, axis=1, keepdims=True))
    o[...] = z / (nrm + 1e-6)


def _tc_dense(hf, haa, hab, w, b, g, bt):
    grid = (N_NODES // _RB,)
    row_spec = pl.BlockSpec((_RB, D_IN), lambda i: (i, 0))
    vec_spec = pl.BlockSpec((1, D_OUT), lambda i: (0, 0))
    return pl.pallas_call(
        _tc_dense_body,
        grid=grid,
        in_specs=[row_spec, row_spec, row_spec,
                  pl.BlockSpec((D_OUT, 2 * D_IN), lambda i: (0, 0)),
                  vec_spec, vec_spec, vec_spec],
        out_specs=pl.BlockSpec((_RB, D_OUT), lambda i: (i, 0)),
        out_shape=jax.ShapeDtypeStruct((N_NODES, D_OUT), jnp.float32),
    )(hf, haa, hab, w, b, g, bt)


def kernel(features, batch, edge_index, W, b, gamma, beta):
    f32 = jnp.float32
    i32 = jnp.int32
    padcols = URPAD * U - EPT
    per_tile_src = edge_index[0].astype(i32).reshape(NC * NS, EPT)
    per_tile_tgt = edge_index[1].astype(i32).reshape(NC * NS, EPT)
    src_r = jnp.pad(per_tile_src, ((0, 0), (0, padcols))).reshape(
        NC, NS, URPAD, U)
    tgt_r = jnp.pad(per_tile_tgt, ((0, 0), (0, padcols))).reshape(
        NC, NS, URPAD, U)
    bat = batch.astype(i32)
    bat_main = bat[:NBCH * U].reshape(NBCH, U)
    bat_tail = bat[NBCH * U:]
    hf, haa, hab = _sc_agg_gather(features, tgt_r, src_r, bat_main, bat_tail)
    return _tc_dense(hf, haa, hab,
                     W.astype(f32), b.reshape(1, D_OUT).astype(f32),
                     gamma.reshape(1, D_OUT).astype(f32),
                     beta.reshape(1, D_OUT).astype(f32))
